# Initial kernel scaffold; baseline (speedup 1.0000x reference)
import jax
import jax.numpy as jnp
from jax.experimental import pallas as pl


def kernel(x, a):
    raise NotImplementedError("write your pallas kernel here")


# speedup vs baseline: 2106.9078x; 2106.9078x over previous
import jax
import jax.numpy as jnp
from jax.experimental import pallas as pl


def _mm_kernel(a_ref, x_ref, o_ref):
    o_ref[...] = jnp.dot(a_ref[...], x_ref[...],
                         preferred_element_type=jnp.float32)


def kernel(x, a):
    n_t, n_s = a.shape
    d = x.shape[1]
    bm = 256
    return pl.pallas_call(
        _mm_kernel,
        grid=(n_t // bm,),
        in_specs=[
            pl.BlockSpec((bm, n_s), lambda i: (i, 0)),
            pl.BlockSpec((n_s, d), lambda i: (0, 0)),
        ],
        out_specs=pl.BlockSpec((bm, d), lambda i: (i, 0)),
        out_shape=jax.ShapeDtypeStruct((n_t, d), jnp.float32),
    )(a, x)


# speedup vs baseline: 2418.7580x; 1.1480x over previous
import jax
import jax.numpy as jnp
from jax.experimental import pallas as pl


def _mm_kernel(a_ref, x_ref, o_ref):
    o_ref[...] = jnp.dot(a_ref[...], x_ref[...],
                         preferred_element_type=jnp.float32)


def kernel(x, a):
    n_t, n_s = a.shape
    d = x.shape[1]
    bm = 512
    return pl.pallas_call(
        _mm_kernel,
        grid=(n_t // bm,),
        in_specs=[
            pl.BlockSpec((bm, n_s), lambda i: (i, 0)),
            pl.BlockSpec((n_s, d), lambda i: (0, 0)),
        ],
        out_specs=pl.BlockSpec((bm, d), lambda i: (i, 0)),
        out_shape=jax.ShapeDtypeStruct((n_t, d), jnp.float32),
    )(a, x)


# speedup vs baseline: 2514.7737x; 1.0397x over previous
import jax
import jax.numpy as jnp
from jax.experimental import pallas as pl


def _mm_kernel(a_ref, x_ref, o_ref):
    o_ref[...] = jnp.dot(a_ref[...], x_ref[...],
                         preferred_element_type=jnp.float32)


def kernel(x, a):
    n_t, n_s = a.shape
    d = x.shape[1]
    bm = 1024
    return pl.pallas_call(
        _mm_kernel,
        grid=(n_t // bm,),
        in_specs=[
            pl.BlockSpec((bm, n_s), lambda i: (i, 0)),
            pl.BlockSpec((n_s, d), lambda i: (0, 0)),
        ],
        out_specs=pl.BlockSpec((bm, d), lambda i: (i, 0)),
        out_shape=jax.ShapeDtypeStruct((n_t, d), jnp.float32),
    )(a, x)
